# baseline (speedup 1.0000x reference)
import functools

import jax
import jax.numpy as jnp
from jax import lax
from jax.experimental import pallas as pl
from jax.experimental.pallas import tpu as pltpu
from jax.experimental.pallas import tpu_sc as plsc

NUM_SEGMENTS = 10000
D = 128
E = 320000
NC = 2
NS = 16
NW = NC * NS
EPW = E // NW
CHUNK = 40
NCHUNK = EPW // CHUNK
NBUF = 8
LOOKAHEAD = 6
ROWS_MAIN = 624
ROWS_LAST_AT = ROWS_MAIN * (NS - 1)
ROWS_LAST = NUM_SEGMENTS - ROWS_LAST_AT
ZB_ROWS = 48


def _sc_body(x_hbm, idx_hbm, out_hbm,
             xbufs, ibufs, zbuf, xsems, isems, ssems, shared):
  c = lax.axis_index("c")
  s = lax.axis_index("s")
  wid = c * NS + s
  base = wid * EPW

  def start_load(k, b):
    pltpu.async_copy(x_hbm.at[pl.ds(base + k * CHUNK, CHUNK)], xbufs[b],
                     xsems[b])
    pltpu.async_copy(idx_hbm.at[pl.ds(base + k * CHUNK, CHUNK)], ibufs[b],
                     isems[b])

  def wait_load(k, b):
    pltpu.make_async_copy(x_hbm.at[pl.ds(base + k * CHUNK, CHUNK)], xbufs[b],
                          xsems[b]).wait()
    pltpu.make_async_copy(idx_hbm.at[pl.ds(base + k * CHUNK, CHUNK)],
                          ibufs[b], isems[b]).wait()

  def start_scatter(b):
    pltpu.async_copy(xbufs[b], shared.at[ibufs[b]], ssems[b], add=True)

  def wait_scatter(b):
    pltpu.make_async_copy(xbufs[b], shared.at[ibufs[b]],
                          ssems[b]).wait()

  for b in range(LOOKAHEAD):
    start_load(b, b)

  zrow = s * ROWS_MAIN
  zv = jnp.zeros((16,), jnp.float32)

  @functools.partial(lax.fori_loop, 0, ZB_ROWS, init_val=None)
  def _(r, _):
    for g in range(D // 16):
      zbuf[r, pl.ds(g * 16, 16)] = zv

  @functools.partial(lax.fori_loop, 0, ROWS_MAIN // ZB_ROWS, init_val=None)
  def _(t, _):
    pltpu.sync_copy(zbuf, shared.at[pl.ds(zrow + t * ZB_ROWS, ZB_ROWS)])

  @pl.when(s == NS - 1)
  def _():
    pltpu.sync_copy(zbuf.at[pl.ds(0, ROWS_LAST - ROWS_MAIN)],
                    shared.at[pl.ds(ROWS_LAST_AT + ROWS_MAIN,
                                    ROWS_LAST - ROWS_MAIN)])

  plsc.subcore_barrier()

  def step(k, b):
    b2 = (b + LOOKAHEAD) % NBUF

    @pl.when(k + LOOKAHEAD < NCHUNK)
    def _():
      @pl.when(k + LOOKAHEAD >= NBUF)
      def _():
        wait_scatter(b2)

      start_load(k + LOOKAHEAD, b2)

    wait_load(k, b)
    start_scatter(b)

  @functools.partial(lax.fori_loop, 0, NCHUNK // NBUF, init_val=None)
  def _(k4, _):
    for b in range(NBUF):
      step(k4 * NBUF + b, b)

  for k in range(NCHUNK - NCHUNK % NBUF, NCHUNK):
    step(k, k % NBUF)

  for b in range(NBUF):
    wait_scatter(b)

  plsc.subcore_barrier()

  @pl.when(s < NS - 1)
  def _():
    pltpu.sync_copy(shared.at[pl.ds(zrow, ROWS_MAIN)],
                    out_hbm.at[c, pl.ds(zrow, ROWS_MAIN)])

  @pl.when(s == NS - 1)
  def _():
    pltpu.sync_copy(shared.at[pl.ds(ROWS_LAST_AT, ROWS_LAST)],
                    out_hbm.at[c, pl.ds(ROWS_LAST_AT, ROWS_LAST)])


def _combine_body(p_ref, o_ref):
  o_ref[...] = p_ref[0] + p_ref[1]


def kernel(x, node_idx):
  mesh = plsc.VectorSubcoreMesh(core_axis_name="c", subcore_axis_name="s")
  partial = pl.kernel(
      _sc_body,
      out_type=jax.ShapeDtypeStruct((NC, NUM_SEGMENTS, D), x.dtype),
      mesh=mesh,
      scratch_types=[
          tuple(pltpu.VMEM((CHUNK, D), jnp.float32) for _ in range(NBUF)),
          tuple(pltpu.VMEM((CHUNK,), jnp.int32) for _ in range(NBUF)),
          pltpu.VMEM((ZB_ROWS, D), jnp.float32),
          tuple(pltpu.SemaphoreType.DMA for _ in range(NBUF)),
          tuple(pltpu.SemaphoreType.DMA for _ in range(NBUF)),
          tuple(pltpu.SemaphoreType.DMA for _ in range(NBUF)),
          pltpu.VMEM_SHARED((NUM_SEGMENTS, D), jnp.float32),
      ],
  )(x, node_idx)

  out = pl.pallas_call(
      _combine_body,
      out_shape=jax.ShapeDtypeStruct((NUM_SEGMENTS, D), x.dtype),
  )(partial)
  return out


# speedup vs baseline: 1.0055x; 1.0055x over previous
import functools

import jax
import jax.numpy as jnp
from jax import lax
from jax.experimental import pallas as pl
from jax.experimental.pallas import tpu as pltpu
from jax.experimental.pallas import tpu_sc as plsc

NUM_SEGMENTS = 10000
D = 128
E = 320000
NC = 2
NS = 16
NW = NC * NS
EPW = E // NW
CHUNK = 40
NCHUNK = EPW // CHUNK
NBUF = 8
LOOKAHEAD = 6
ROWS_MAIN = 624
ROWS_LAST_AT = ROWS_MAIN * (NS - 1)
ROWS_LAST = NUM_SEGMENTS - ROWS_LAST_AT
ZB_ROWS = 48


def _sc_body(x_hbm, idx_hbm, out_hbm,
             xbufs, ibufs, zbuf, xsems, isems, ssems, shared):
  c = lax.axis_index("c")
  s = lax.axis_index("s")
  wid = s * NC + c
  base = wid * EPW

  def start_load(k, b):
    pltpu.async_copy(x_hbm.at[pl.ds(base + k * CHUNK, CHUNK)], xbufs[b],
                     xsems[b])
    pltpu.async_copy(idx_hbm.at[pl.ds(base + k * CHUNK, CHUNK)], ibufs[b],
                     isems[b])

  def wait_load(k, b):
    pltpu.make_async_copy(x_hbm.at[pl.ds(base + k * CHUNK, CHUNK)], xbufs[b],
                          xsems[b]).wait()
    pltpu.make_async_copy(idx_hbm.at[pl.ds(base + k * CHUNK, CHUNK)],
                          ibufs[b], isems[b]).wait()

  def start_scatter(b):
    pltpu.async_copy(xbufs[b], shared.at[ibufs[b]], ssems[b], add=True)

  def wait_scatter(b):
    pltpu.make_async_copy(xbufs[b], shared.at[ibufs[b]],
                          ssems[b]).wait()

  for b in range(LOOKAHEAD):
    start_load(b, b)

  zrow = s * ROWS_MAIN
  zv = jnp.zeros((16,), jnp.float32)

  @functools.partial(lax.fori_loop, 0, ZB_ROWS, init_val=None)
  def _(r, _):
    for g in range(D // 16):
      zbuf[r, pl.ds(g * 16, 16)] = zv

  @functools.partial(lax.fori_loop, 0, ROWS_MAIN // ZB_ROWS, init_val=None)
  def _(t, _):
    pltpu.sync_copy(zbuf, shared.at[pl.ds(zrow + t * ZB_ROWS, ZB_ROWS)])

  @pl.when(s == NS - 1)
  def _():
    pltpu.sync_copy(zbuf.at[pl.ds(0, ROWS_LAST - ROWS_MAIN)],
                    shared.at[pl.ds(ROWS_LAST_AT + ROWS_MAIN,
                                    ROWS_LAST - ROWS_MAIN)])

  plsc.subcore_barrier()

  def step(k, b):
    b2 = (b + LOOKAHEAD) % NBUF

    @pl.when(k + LOOKAHEAD < NCHUNK)
    def _():
      @pl.when(k + LOOKAHEAD >= NBUF)
      def _():
        wait_scatter(b2)

      start_load(k + LOOKAHEAD, b2)

    wait_load(k, b)
    start_scatter(b)

  @functools.partial(lax.fori_loop, 0, NCHUNK // NBUF, init_val=None)
  def _(k4, _):
    for b in range(NBUF):
      step(k4 * NBUF + b, b)

  for k in range(NCHUNK - NCHUNK % NBUF, NCHUNK):
    step(k, k % NBUF)

  for b in range(NBUF):
    wait_scatter(b)

  plsc.subcore_barrier()

  @pl.when(s < NS - 1)
  def _():
    pltpu.sync_copy(shared.at[pl.ds(zrow, ROWS_MAIN)],
                    out_hbm.at[c, pl.ds(zrow, ROWS_MAIN)])

  @pl.when(s == NS - 1)
  def _():
    pltpu.sync_copy(shared.at[pl.ds(ROWS_LAST_AT, ROWS_LAST)],
                    out_hbm.at[c, pl.ds(ROWS_LAST_AT, ROWS_LAST)])


def _combine_body(p_ref, o_ref):
  o_ref[...] = p_ref[0] + p_ref[1]


def kernel(x, node_idx):
  mesh = plsc.VectorSubcoreMesh(core_axis_name="c", subcore_axis_name="s")
  partial = pl.kernel(
      _sc_body,
      out_type=jax.ShapeDtypeStruct((NC, NUM_SEGMENTS, D), x.dtype),
      mesh=mesh,
      scratch_types=[
          tuple(pltpu.VMEM((CHUNK, D), jnp.float32) for _ in range(NBUF)),
          tuple(pltpu.VMEM((CHUNK,), jnp.int32) for _ in range(NBUF)),
          pltpu.VMEM((ZB_ROWS, D), jnp.float32),
          tuple(pltpu.SemaphoreType.DMA for _ in range(NBUF)),
          tuple(pltpu.SemaphoreType.DMA for _ in range(NBUF)),
          tuple(pltpu.SemaphoreType.DMA for _ in range(NBUF)),
          pltpu.VMEM_SHARED((NUM_SEGMENTS, D), jnp.float32),
      ],
  )(x, node_idx)

  out = pl.pallas_call(
      _combine_body,
      out_shape=jax.ShapeDtypeStruct((NUM_SEGMENTS, D), x.dtype),
  )(partial)
  return out
